# baseline (speedup 1.0000x reference)
import jax
import jax.numpy as jnp
from jax import lax
from jax.experimental import pallas as pl
from jax.experimental.pallas import tpu as pltpu
from jax.experimental.pallas import tpu_sc as plsc

NUM_ENTITIES = 1000000
NUM_RELATIONS = 1000
EMBED_DIM = 32
BATCH = 16384
PER_ROW = 128 // EMBED_DIM

NC = 2
NS = 16
LANES = 16
NW = NC * NS
BPW = BATCH // NW
CHUNK = 128
NCHUNK = BPW // CHUNK


def _body(heads_hbm, rels_hbm, tails_hbm, ent_hbm, rel_hbm, out_hbm,
          vidx_h, vidx_t, vidx_r, qidx_h, qidx_t,
          hbuf, tbuf, rel_v, out_v, sem, ssem):
    wid = lax.axis_index("s") * NC + lax.axis_index("c")
    base = wid * BPW

    for j in range(NCHUNK):
        src = pl.ds(base + j * CHUNK, CHUNK)
        pltpu.sync_copy(heads_hbm.at[src], vidx_h.at[j])
        pltpu.sync_copy(tails_hbm.at[src], vidx_t.at[j])
        pltpu.sync_copy(rels_hbm.at[src], vidx_r.at[j])
        for k in range(CHUNK // LANES):
            sl = pl.ds(k * LANES, LANES)
            qidx_h[j, sl] = lax.shift_right_logical(vidx_h[j, sl], 2)
            qidx_t[j, sl] = lax.shift_right_logical(vidx_t[j, sl], 2)

    rel_copy = pltpu.async_copy(rel_hbm, rel_v, ssem)

    lane = lax.iota(jnp.int32, LANES)
    rel_copy.wait()

    def chunk_body(c, carry):
        hc = pltpu.async_copy(ent_hbm.at[qidx_h.at[c]], hbuf, sem)
        tc = pltpu.async_copy(ent_hbm.at[qidx_t.at[c]], tbuf, sem)
        hc.wait()
        tc.wait()

        def group(g, carry2):
            sl = pl.ds(g * LANES, LANES)
            li = lane + g * LANES
            hoff = (vidx_h[c, sl] & (PER_ROW - 1)) * EMBED_DIM
            toff = (vidx_t[c, sl] & (PER_ROW - 1)) * EMBED_DIM
            ridx = vidx_r[c, sl]
            rq = lax.shift_right_logical(ridx, 2)
            roff = (ridx & (PER_ROW - 1)) * EMBED_DIM
            acc = jnp.zeros((LANES,), jnp.float32)
            for d in range(EMBED_DIM):
                hv = plsc.load_gather(hbuf, [li, hoff + d])
                tv = plsc.load_gather(tbuf, [li, toff + d])
                rv = plsc.load_gather(rel_v, [rq, roff + d])
                acc = acc + hv * rv * tv
            out_v[pl.ds(c * CHUNK + g * LANES, LANES)] = acc
            return carry2

        lax.fori_loop(0, CHUNK // LANES, group, 0)
        return carry

    lax.fori_loop(0, NCHUNK, chunk_body, 0)

    pltpu.sync_copy(out_v, out_hbm.at[pl.ds(base, BPW)])


@jax.jit
def _run(heads, relations, tails, ent4, rel4):
    mesh = plsc.VectorSubcoreMesh(core_axis_name="c", subcore_axis_name="s")
    k = pl.kernel(
        _body,
        out_type=jax.ShapeDtypeStruct((BATCH,), jnp.float32),
        mesh=mesh,
        compiler_params=pltpu.CompilerParams(needs_layout_passes=False),
        scratch_types=[
            pltpu.VMEM((NCHUNK, CHUNK), jnp.int32),
            pltpu.VMEM((NCHUNK, CHUNK), jnp.int32),
            pltpu.VMEM((NCHUNK, CHUNK), jnp.int32),
            pltpu.VMEM((NCHUNK, CHUNK), jnp.int32),
            pltpu.VMEM((NCHUNK, CHUNK), jnp.int32),
            pltpu.VMEM((CHUNK, 128), jnp.float32),
            pltpu.VMEM((CHUNK, 128), jnp.float32),
            pltpu.VMEM((NUM_RELATIONS // PER_ROW, 128), jnp.float32),
            pltpu.VMEM((BPW,), jnp.float32),
            pltpu.SemaphoreType.DMA,
            pltpu.SemaphoreType.DMA,
        ],
    )
    return k(heads, relations, tails, ent4, rel4)


def kernel(heads, relations, tails, entity_table, relation_table):
    return _run(
        heads.astype(jnp.int32),
        relations.astype(jnp.int32),
        tails.astype(jnp.int32),
        lax.optimization_barrier(entity_table.reshape(NUM_ENTITIES // PER_ROW, 128)),
        lax.optimization_barrier(relation_table.reshape(NUM_RELATIONS // PER_ROW, 128)),
    )
